# baseline (speedup 1.0000x reference)
import functools

import jax
import jax.numpy as jnp
from jax import lax
from jax.experimental import pallas as pl
from jax.experimental.pallas import tpu as pltpu
from jax.experimental.pallas import tpu_sc as plsc

_ALPHA = 0.2
_NEG = -9e15


def _sc_nbr_call(flat, adjp, nump):
    F = flat.shape[0]
    N, SP = adjp.shape
    info = plsc.get_sparse_core_info()
    NC, NS = info.num_cores, info.num_subcores
    NW = NC * NS
    FW = F // NW
    mesh = plsc.VectorSubcoreMesh(core_axis_name="c", subcore_axis_name="s")

    @functools.partial(
        pl.kernel,
        out_type=(
            jax.ShapeDtypeStruct((F, SP), jnp.int32),
            jax.ShapeDtypeStruct((F, SP), jnp.float32),
        ),
        mesh=mesh,
        compiler_params=pltpu.CompilerParams(use_tc_tiling_on_sc=False),
        scratch_types=[
            pltpu.VMEM((FW,), jnp.int32),
            pltpu.VMEM((FW, SP), jnp.int32),
            pltpu.VMEM((FW, SP), jnp.float32),
            pltpu.SemaphoreType.DMA,
            pltpu.SemaphoreType.DMA,
        ],
    )
    def sc_a(flat_hbm, adjp_hbm, nump_hbm, ids_out, w_out,
             ids_v, nbr_v, wv_v, sema, semb):
        wid = lax.axis_index("s") * NC + lax.axis_index("c")
        base = wid * FW
        pltpu.sync_copy(flat_hbm.at[pl.ds(base, FW)], ids_v)
        cpn = pltpu.async_copy(adjp_hbm.at[ids_v], nbr_v, sema)
        cpw = pltpu.async_copy(nump_hbm.at[ids_v], wv_v, semb)
        cpn.wait()
        pltpu.sync_copy(nbr_v, ids_out.at[pl.ds(base, FW)])
        cpw.wait()
        pltpu.sync_copy(wv_v, w_out.at[pl.ds(base, FW)])

    return sc_a(flat, adjp, nump)


def _sc_emb_call(ids_all, embedding):
    T = ids_all.shape[0]
    D = embedding.shape[1]
    info = plsc.get_sparse_core_info()
    NC, NS = info.num_cores, info.num_subcores
    NW = NC * NS
    TW = T // NW
    NCH = 30
    CH = TW // NCH
    mesh = plsc.VectorSubcoreMesh(core_axis_name="c", subcore_axis_name="s")

    @functools.partial(
        pl.kernel,
        out_type=jax.ShapeDtypeStruct((T, D), jnp.float32),
        mesh=mesh,
        compiler_params=pltpu.CompilerParams(use_tc_tiling_on_sc=False),
        scratch_types=[
            pltpu.VMEM((TW,), jnp.int32),
            pltpu.VMEM((4, CH, D), jnp.float32),
            pltpu.SemaphoreType.DMA,
            pltpu.SemaphoreType.DMA,
            pltpu.SemaphoreType.DMA,
            pltpu.SemaphoreType.DMA,
            pltpu.SemaphoreType.DMA,
            pltpu.SemaphoreType.DMA,
            pltpu.SemaphoreType.DMA,
            pltpu.SemaphoreType.DMA,
        ],
    )
    def sc_b(ids_hbm, emb_hbm, rows_out,
             idx_v, bufs, sg0, sg1, sg2, sg3, sw0, sw1, sw2, sw3):
        wid = lax.axis_index("s") * NC + lax.axis_index("c")
        base = wid * TW
        gsems = [sg0, sg1, sg2, sg3]
        wsems = [sw0, sw1, sw2, sw3]
        pltpu.sync_copy(ids_hbm.at[pl.ds(base, TW)], idx_v)
        gcps = [None] * NCH
        wcps = [None] * NCH
        for c in range(NCH + 1):
            if c < NCH:
                s = c % 4
                if c >= 4:
                    wcps[c - 4].wait()
                gcps[c] = pltpu.async_copy(
                    emb_hbm.at[idx_v.at[pl.ds(c * CH, CH)]], bufs.at[s],
                    gsems[s])
            if c >= 1:
                pc = c - 1
                gcps[pc].wait()
                wcps[pc] = pltpu.async_copy(
                    bufs.at[pc % 4], rows_out.at[pl.ds(base + pc * CH, CH)],
                    wsems[pc % 4])
        for c in range(max(0, NCH - 4), NCH):
            wcps[c].wait()

    return sc_b(ids_all, embedding)


_NEG2 = -1.8e16


def _tc_call(hf, mf, msn, itf, wq, h1f, AT, gw1, w2r, gw3, Ex, B, L, SP, nS):
    D = hf.shape[1]
    BB = 16
    M = BB * L
    G = B // BB

    def body(h_ref, mf_ref, msn_ref, it_ref, wq_ref, h1_ref,
             A_ref, w1_ref, w2_ref, w3_ref, Ex_ref, hid_ref, gl_ref):
        h = h_ref[...]
        mfb = mf_ref[...].reshape(M, M)
        att = jnp.where(mfb >= 1, _NEG, _NEG2)
        for k in range(4):
            q = h * A_ref[k, :][None, :]
            e = lax.dot_general(q, h, (((1,), (1,)), ((), ())),
                                preferred_element_type=jnp.float32)
            e = jnp.maximum(e, _ALPHA * e)
            att = jnp.where(mfb == (k + 2), e, att)
        att = att - jnp.max(att, axis=-1, keepdims=True)
        p = jnp.exp(att)
        att = p / jnp.sum(p, axis=-1, keepdims=True)
        hid_ref[...] = lax.dot_general(att, h, (((1,), (0,)), ((), ())),
                                       preferred_element_type=jnp.float32)

        sess = lax.dot_general(msn_ref[...].reshape(BB, M), it_ref[...],
                               (((1,), (0,)), ((), ())),
                               preferred_element_type=jnp.float32)
        srow = lax.dot_general(Ex_ref[...], sess, (((1,), (0,)), ((), ())),
                               preferred_element_type=jnp.float32)

        h1 = h1_ref[...]
        t = lax.dot_general(h1 * srow, w1_ref[...][:D, :],
                            (((1,), (0,)), ((), ())),
                            preferred_element_type=jnp.float32)
        t3 = t.reshape(M, SP, D)
        t3 = t3 + wq_ref[...][..., None] * w1_ref[...][D, :][None, None, :]
        t3 = jnp.maximum(t3, _ALPHA * t3)
        s = jnp.sum(t3 * w2_ref[...][0, :][None, None, :], axis=-1)
        lane = lax.broadcasted_iota(jnp.int32, (M, SP), 1)
        s = jnp.where(lane < nS, s, _NEG2)
        s = s - jnp.max(s, axis=-1, keepdims=True)
        es = jnp.exp(s)
        a = es / jnp.sum(es, axis=-1, keepdims=True)
        nv = jnp.sum(a[..., None] * h1.reshape(M, SP, D), axis=1)
        cat = jnp.concatenate([h, nv], axis=-1)
        out = lax.dot_general(cat, w3_ref[...], (((1,), (0,)), ((), ())),
                              preferred_element_type=jnp.float32)
        gl_ref[...] = jnp.maximum(out, 0.0)

    bspec = lambda shp: pl.BlockSpec(shp, lambda i: (i,) + (0,) * (len(shp) - 1))
    full = lambda arr: pl.BlockSpec(arr.shape, lambda i: (0,) * arr.ndim)
    return pl.pallas_call(
        body,
        grid=(G,),
        in_specs=[
            bspec((M, D)),
            bspec((1, M, M)),
            bspec((1, BB, M)),
            bspec((M, D)),
            bspec((M, SP)),
            bspec((M * SP, D)),
            full(AT), full(gw1), full(w2r), full(gw3), full(Ex),
        ],
        out_specs=[bspec((M, D)), bspec((M, D))],
        out_shape=[
            jax.ShapeDtypeStruct((B * L, D), jnp.float32),
            jax.ShapeDtypeStruct((B * L, D), jnp.float32),
        ],
    )(hf, mf, msn, itf, wq, h1f, AT, gw1, w2r, gw3, Ex)


def kernel(inputs, adj, mask_item, item, adj_all, num, embedding,
           a0, a1, a2, a3, gw1, gw2, gw3):
    B, L = inputs.shape
    N, S = adj_all.shape
    D = embedding.shape[1]
    SP = 16
    flat = inputs.reshape(-1).astype(jnp.int32)
    itf = item.reshape(-1).astype(jnp.int32)
    adjp = jnp.concatenate(
        [adj_all.astype(jnp.int32), jnp.zeros((N, SP - S), jnp.int32)], axis=1)
    nump = jnp.concatenate([num, jnp.zeros((N, SP - S), num.dtype)], axis=1)
    ids16, w_rows = _sc_nbr_call(flat, adjp, nump)
    ids_flat = ids16.reshape(-1)
    F = flat.shape[0]
    R = ids_flat.shape[0]
    rows = _sc_emb_call(jnp.concatenate([ids_flat, flat, itf]), embedding)
    h1 = rows[:R]
    h_rows = rows[R:R + F]
    it_rows = rows[R + F:]
    BB = 16
    G = B // BB
    M = BB * L
    adj5 = adj.astype(jnp.int32).reshape(G, BB, 1, L, L) + 1
    eye = jnp.eye(BB, dtype=jnp.bool_)[None, :, :, None, None]
    mf = jnp.where(eye, adj5, 0)
    mf = mf.transpose(0, 1, 3, 2, 4).reshape(G, M, M)
    m3 = mask_item.reshape(G, BB, L)
    mn = m3 / jnp.sum(m3, axis=2, keepdims=True)
    eye2 = jnp.eye(BB, dtype=jnp.bool_)[None, :, :, None]
    msn = jnp.where(eye2, mn[:, :, None, :], 0.0).reshape(G, BB, M)
    rr = jnp.arange(M * SP, dtype=jnp.int32) // (L * SP)
    Ex = (rr[:, None] == jnp.arange(BB, dtype=jnp.int32)[None, :]).astype(
        jnp.float32)
    AT = jnp.concatenate([a0, a1, a2, a3], axis=1).T
    hid, glob = _tc_call(
        h_rows, mf, msn, it_rows, w_rows, h1,
        AT, gw1, gw2.T, gw3, Ex, B, L, SP, S)
    return hid.reshape(B, L, D), glob.reshape(B, L, D)
